# baseline (device time: 126526 ns/iter reference)
import jax
import jax.numpy as jnp
from jax import lax
from jax.experimental import pallas as pl
from jax.experimental.pallas import tpu as pltpu

N_DEV = 4
B = 2
SQ = 512
SKV = 512
HG = 2048
HL = 512
NH = 8
DH = 64
DM = 768
ROWS = B * SQ
BLK = ROWS // N_DEV
GW = 32
GR = B * GW
BAND = 640
BW = BAND - SKV
NEG = -1e9


def kernel(x, Wq, K_ext, V_ext, Wo):
    xb = x.reshape(ROWS, DM).astype(jnp.bfloat16)
    wqb = Wq.astype(jnp.bfloat16)
    wob = Wo.astype(jnp.bfloat16)
    kb = K_ext.reshape(B, SKV, HG).astype(jnp.bfloat16)
    vb = V_ext.reshape(B, SKV, HG).astype(jnp.bfloat16)

    def body(x_ref, wq_ref, k_ref, v_ref, wo_ref, out_ref,
             kown, vown, kg0, vg0, kband, vband,
             qb, qsnd, qlg, ctxp, mlp, ctxg, mlg,
             cb, pb, pbb, rsb, ags, agb,
             lsem, ks0, vs0, kr0, vr0, kbs, vbs, kbr, vbr,
             qs, qr, cps, cpr, mls, mlr,
             rss, rsr, agss, agr):
        my = lax.axis_index("i")

        dko = pltpu.make_async_copy(k_ref, kown, lsem.at[0])
        dvo = pltpu.make_async_copy(v_ref, vown, lsem.at[1])
        dko.start()
        dvo.start()

        bsem = pltpu.get_barrier_semaphore()
        for d in range(1, N_DEV):
            pl.semaphore_signal(
                bsem, inc=1,
                device_id=(lax.rem(my + d, N_DEV),),
                device_id_type=pl.DeviceIdType.MESH)
        pl.semaphore_wait(bsem, N_DEV - 1)

        kv0_sends = []
        for t in (1, 2, 3):
            rk = pltpu.make_async_remote_copy(
                src_ref=k_ref.at[:, :, pl.ds(t * HL, HL)], dst_ref=kg0,
                send_sem=ks0.at[t], recv_sem=kr0.at[0],
                device_id=(t,), device_id_type=pl.DeviceIdType.MESH)
            rv = pltpu.make_async_remote_copy(
                src_ref=v_ref.at[:, :, pl.ds(t * HL, HL)], dst_ref=vg0,
                send_sem=vs0.at[t], recv_sem=vr0.at[0],
                device_id=(t,), device_id_type=pl.DeviceIdType.MESH)
            kv0_sends += [rk, rv]
        band_sends = []
        for t in (0, 2, 3):
            rk = pltpu.make_async_remote_copy(
                src_ref=k_ref.at[:, pl.ds(0, BW), pl.ds(t * HL, HL)],
                dst_ref=kband,
                send_sem=kbs.at[t], recv_sem=kbr.at[0],
                device_id=(t,), device_id_type=pl.DeviceIdType.MESH)
            rv = pltpu.make_async_remote_copy(
                src_ref=v_ref.at[:, pl.ds(0, BW), pl.ds(t * HL, HL)],
                dst_ref=vband,
                send_sem=vbs.at[t], recv_sem=vbr.at[0],
                device_id=(t,), device_id_type=pl.DeviceIdType.MESH)
            band_sends += [rk, rv]

        @pl.when(my == 0)
        def _():
            for r in kv0_sends:
                r.start()

        @pl.when(my == 1)
        def _():
            for r in band_sends:
                r.start()

        qb[...] = jnp.dot(x_ref[...], wq_ref[...],
                          preferred_element_type=jnp.float32
                          ).astype(jnp.bfloat16)

        qsnd[0:GW, :] = qb[0:GW, :]
        qsnd[GW:GR, :] = qb[SQ:SQ + GW, :]
        sends = []
        for d in range(1, N_DEV):
            tgt = lax.rem(my + d, N_DEV)
            r = pltpu.make_async_remote_copy(
                src_ref=qsnd, dst_ref=qlg.at[my],
                send_sem=qs.at[tgt], recv_sem=qr.at[my],
                device_id=(tgt,), device_id_type=pl.DeviceIdType.MESH)
            r.start()
            sends.append(r)
        qlg[my] = qsnd[...]

        dko.wait()
        dvo.wait()

        @pl.when(my == 0)
        def _():
            kg0[...] = kown[:, :, 0:HL]
            vg0[...] = vown[:, :, 0:HL]

        @pl.when(my == 1)
        def _():
            kband[...] = kown[:, 0:BW, HL:2 * HL]
            vband[...] = vown[:, 0:BW, HL:2 * HL]

        for d in range(1, N_DEV):
            src = lax.rem(my + d, N_DEV)
            pltpu.make_async_remote_copy(
                src_ref=qsnd, dst_ref=qlg.at[src],
                send_sem=qs.at[src], recv_sem=qr.at[src],
                device_id=(src,),
                device_id_type=pl.DeviceIdType.MESH).wait_recv()

        for s in range(N_DEV):
            for b in range(B):
                mparts = []
                lparts = []
                for hh in range(NH):
                    g = s * NH + hh
                    q = qlg[s, b * GW:(b + 1) * GW, hh * DH:(hh + 1) * DH]
                    k = kown[b, :, g * DH:(g + 1) * DH]
                    v = vown[b, :, g * DH:(g + 1) * DH]
                    sc = lax.dot_general(
                        q, k, (((1,), (1,)), ((), ())),
                        preferred_element_type=jnp.float32) * 0.125
                    m = jnp.max(sc, axis=1, keepdims=True)
                    e = jnp.exp(sc - m)
                    l = jnp.sum(e, axis=1, keepdims=True)
                    ctxp[s, b * GW:(b + 1) * GW, hh * DH:(hh + 1) * DH] = (
                        lax.dot_general(
                            e.astype(jnp.bfloat16), v,
                            (((1,), (0,)), ((), ())),
                            preferred_element_type=jnp.float32
                        ).astype(jnp.bfloat16))
                    mparts.append(m)
                    lparts.append(l)
                mlp[s, b * GW:(b + 1) * GW, 0:NH] = jnp.concatenate(
                    mparts, axis=1)
                mlp[s, b * GW:(b + 1) * GW, NH:2 * NH] = jnp.concatenate(
                    lparts, axis=1)

        for d in range(1, N_DEV):
            tgt = lax.rem(my + d, N_DEV)
            rc = pltpu.make_async_remote_copy(
                src_ref=ctxp.at[tgt], dst_ref=ctxg.at[my],
                send_sem=cps.at[tgt], recv_sem=cpr.at[my],
                device_id=(tgt,), device_id_type=pl.DeviceIdType.MESH)
            rm = pltpu.make_async_remote_copy(
                src_ref=mlp.at[tgt], dst_ref=mlg.at[my],
                send_sem=mls.at[tgt], recv_sem=mlr.at[my],
                device_id=(tgt,), device_id_type=pl.DeviceIdType.MESH)
            rc.start()
            rm.start()
            sends += [rc, rm]
        ctxg[my] = ctxp[my]
        mlg[my] = mlp[my]

        @pl.when(my != 0)
        def _():
            for (gref, rsem) in ((kg0, kr0), (vg0, vr0)):
                pltpu.make_async_remote_copy(
                    src_ref=kg0, dst_ref=gref,
                    send_sem=ks0.at[0], recv_sem=rsem.at[0],
                    device_id=(0,),
                    device_id_type=pl.DeviceIdType.MESH).wait_recv()

        @pl.when(my != 1)
        def _():
            for (gref, rsem) in ((kband, kbr), (vband, vbr)):
                pltpu.make_async_remote_copy(
                    src_ref=kband, dst_ref=gref,
                    send_sem=kbs.at[0], recv_sem=rsem.at[0],
                    device_id=(1,),
                    device_id_type=pl.DeviceIdType.MESH).wait_recv()

        qi = lax.broadcasted_iota(jnp.int32, (SQ, BAND), 0)
        ki = lax.broadcasted_iota(jnp.int32, (SQ, BAND), 1)
        mask = (jnp.abs(qi - ki) <= 128) | (ki < GW) | (qi < GW)

        for b in range(B):
            for h in range(NH):
                cols = pl.ds(h * DH, DH)
                rows = pl.ds(b * SQ, SQ)
                q = qb[rows, cols]
                k = jnp.concatenate(
                    [kg0[b, :, cols], kband[b, :, cols]], axis=0)
                v = jnp.concatenate(
                    [vg0[b, :, cols], vband[b, :, cols]], axis=0)
                s = lax.dot_general(
                    q, k, (((1,), (1,)), ((), ())),
                    preferred_element_type=jnp.float32) * 0.125
                s = jnp.where(mask, s, NEG)
                m = jnp.max(s, axis=1, keepdims=True)
                w = jnp.exp(s - m)
                w = (w / jnp.sum(w, axis=1, keepdims=True)).astype(jnp.bfloat16)
                cb[rows, cols] = lax.dot_general(
                    w, v, (((1,), (0,)), ((), ())),
                    preferred_element_type=jnp.float32).astype(jnp.bfloat16)

        for d in range(1, N_DEV):
            src = lax.rem(my + d, N_DEV)
            pltpu.make_async_remote_copy(
                src_ref=ctxp.at[0], dst_ref=ctxg.at[src],
                send_sem=cps.at[src], recv_sem=cpr.at[src],
                device_id=(src,),
                device_id_type=pl.DeviceIdType.MESH).wait_recv()
            pltpu.make_async_remote_copy(
                src_ref=mlp.at[0], dst_ref=mlg.at[src],
                send_sem=mls.at[src], recv_sem=mlr.at[src],
                device_id=(src,),
                device_id_type=pl.DeviceIdType.MESH).wait_recv()

        for b in range(B):
            rows = pl.ds(b * GW, GW)
            m_all = mlg[:, rows, 0:NH]
            l_all = mlg[:, rows, NH:2 * NH]
            M = jnp.max(m_all, axis=0)
            alpha = jnp.exp(m_all - M[None])
            L = jnp.sum(alpha * l_all, axis=0)
            ar = jnp.broadcast_to(
                alpha[:, :, :, None], (N_DEV, GW, NH, DH)
            ).reshape(N_DEV, GW, HL)
            ctxsum = jnp.sum(
                ar * ctxg[:, rows, :].astype(jnp.float32), axis=0)
            Lr = jnp.broadcast_to(
                L[:, :, None], (GW, NH, DH)).reshape(GW, HL)
            cb[pl.ds(b * SQ, GW), :] = (ctxsum / Lr).astype(jnp.bfloat16)

        pb[...] = jnp.dot(cb[...], wo_ref[...],
                          preferred_element_type=jnp.float32)
        pbb[...] = pb[...].astype(jnp.bfloat16)

        for d in range(1, N_DEV):
            tgt = lax.rem(my + d, N_DEV)
            r = pltpu.make_async_remote_copy(
                src_ref=pbb.at[pl.ds(tgt * BLK, BLK)],
                dst_ref=rsb.at[my],
                send_sem=rss.at[tgt], recv_sem=rsr.at[my],
                device_id=(tgt,), device_id_type=pl.DeviceIdType.MESH)
            r.start()
            sends.append(r)

        acc = pb[pl.ds(my * BLK, BLK), :]
        for d in range(1, N_DEV):
            src = lax.rem(my + d, N_DEV)
            pltpu.make_async_remote_copy(
                src_ref=pbb.at[pl.ds(0, BLK)], dst_ref=rsb.at[src],
                send_sem=rss.at[src], recv_sem=rsr.at[src],
                device_id=(src,),
                device_id_type=pl.DeviceIdType.MESH).wait_recv()
            acc = acc + rsb[src].astype(jnp.float32)
        out_ref[pl.ds(my * BLK, BLK), :] = acc
        ags[...] = acc.astype(jnp.bfloat16)

        for d in range(1, N_DEV):
            tgt = lax.rem(my + d, N_DEV)
            r = pltpu.make_async_remote_copy(
                src_ref=ags, dst_ref=agb.at[my],
                send_sem=agss.at[tgt], recv_sem=agr.at[my],
                device_id=(tgt,), device_id_type=pl.DeviceIdType.MESH)
            r.start()
            sends.append(r)
        for d in range(1, N_DEV):
            src = lax.rem(my + d, N_DEV)
            pltpu.make_async_remote_copy(
                src_ref=ags, dst_ref=agb.at[src],
                send_sem=agss.at[src], recv_sem=agr.at[src],
                device_id=(src,),
                device_id_type=pl.DeviceIdType.MESH).wait_recv()
            out_ref[pl.ds(src * BLK, BLK), :] = agb[src].astype(jnp.float32)

        for r in sends:
            r.wait_send()

        @pl.when(my == 0)
        def _():
            for r in kv0_sends:
                r.wait_send()

        @pl.when(my == 1)
        def _():
            for r in band_sends:
                r.wait_send()

    out = pl.pallas_call(
        body,
        out_shape=jax.ShapeDtypeStruct((ROWS, DM), jnp.float32),
        in_specs=[
            pl.BlockSpec(memory_space=pltpu.VMEM),
            pl.BlockSpec(memory_space=pltpu.VMEM),
            pl.BlockSpec(memory_space=pl.ANY),
            pl.BlockSpec(memory_space=pl.ANY),
            pl.BlockSpec(memory_space=pltpu.VMEM),
        ],
        out_specs=pl.BlockSpec(memory_space=pltpu.VMEM),
        scratch_shapes=[
            pltpu.VMEM((B, SKV, HG), jnp.bfloat16),
            pltpu.VMEM((B, SKV, HG), jnp.bfloat16),
            pltpu.VMEM((B, SKV, HL), jnp.bfloat16),
            pltpu.VMEM((B, SKV, HL), jnp.bfloat16),
            pltpu.VMEM((B, BW, HL), jnp.bfloat16),
            pltpu.VMEM((B, BW, HL), jnp.bfloat16),
            pltpu.VMEM((ROWS, HL), jnp.bfloat16),
            pltpu.VMEM((GR, HL), jnp.bfloat16),
            pltpu.VMEM((N_DEV, GR, HL), jnp.bfloat16),
            pltpu.VMEM((N_DEV, GR, HL), jnp.bfloat16),
            pltpu.VMEM((N_DEV, GR, 2 * NH), jnp.float32),
            pltpu.VMEM((N_DEV, GR, HL), jnp.bfloat16),
            pltpu.VMEM((N_DEV, GR, 2 * NH), jnp.float32),
            pltpu.VMEM((ROWS, HL), jnp.bfloat16),
            pltpu.VMEM((ROWS, DM), jnp.float32),
            pltpu.VMEM((ROWS, DM), jnp.bfloat16),
            pltpu.VMEM((N_DEV, BLK, DM), jnp.bfloat16),
            pltpu.VMEM((BLK, DM), jnp.bfloat16),
            pltpu.VMEM((N_DEV, BLK, DM), jnp.bfloat16),
            pltpu.SemaphoreType.DMA((2,)),
            pltpu.SemaphoreType.DMA((N_DEV,)),
            pltpu.SemaphoreType.DMA((N_DEV,)),
            pltpu.SemaphoreType.DMA((1,)),
            pltpu.SemaphoreType.DMA((1,)),
            pltpu.SemaphoreType.DMA((N_DEV,)),
            pltpu.SemaphoreType.DMA((N_DEV,)),
            pltpu.SemaphoreType.DMA((1,)),
            pltpu.SemaphoreType.DMA((1,)),
            pltpu.SemaphoreType.DMA((N_DEV,)),
            pltpu.SemaphoreType.DMA((N_DEV,)),
            pltpu.SemaphoreType.DMA((N_DEV,)),
            pltpu.SemaphoreType.DMA((N_DEV,)),
            pltpu.SemaphoreType.DMA((N_DEV,)),
            pltpu.SemaphoreType.DMA((N_DEV,)),
            pltpu.SemaphoreType.DMA((N_DEV,)),
            pltpu.SemaphoreType.DMA((N_DEV,)),
            pltpu.SemaphoreType.DMA((N_DEV,)),
            pltpu.SemaphoreType.DMA((N_DEV,)),
        ],
        compiler_params=pltpu.CompilerParams(
            collective_id=0,
            vmem_limit_bytes=100 * 1024 * 1024,
        ),
    )(xb, wqb, kb, vb, wob)
    return out.reshape(B, SQ, DM)
